# initial kernel scaffold (unmeasured)
import jax
import jax.numpy as jnp
from jax import lax
from jax.experimental import pallas as pl
from jax.experimental.pallas import tpu as pltpu

N_DEV = 32


def kernel(A, B):
    m, k = A.shape
    _, n = B.shape
    rows = m // N_DEV

    def body(a_ref, b_ref, out_ref, p_ref, rbuf_ref, send_sems, recv_sems):
        my = lax.axis_index("i")

        p_ref[...] = jnp.dot(
            a_ref[...], b_ref[...], preferred_element_type=jnp.float32
        )

        rbuf_ref[pl.ds(my, 1)] = p_ref[pl.ds(my * rows, rows), :][None]

        sends = []
        for off in range(1, N_DEV):
            d = lax.rem(my + off, N_DEV)
            rdma = pltpu.make_async_remote_copy(
                src_ref=p_ref.at[pl.ds(d * rows, rows), :],
                dst_ref=rbuf_ref.at[my],
                send_sem=send_sems.at[off],
                recv_sem=recv_sems.at[my],
                device_id=(d,),
                device_id_type=pl.DeviceIdType.MESH,
            )
            rdma.start()
            sends.append(rdma)

        for off in range(1, N_DEV):
            s = lax.rem(my + off, N_DEV)
            recv = pltpu.make_async_remote_copy(
                src_ref=p_ref.at[pl.ds(0, rows), :],
                dst_ref=rbuf_ref.at[s],
                send_sem=send_sems.at[0],
                recv_sem=recv_sems.at[s],
                device_id=(s,),
                device_id_type=pl.DeviceIdType.MESH,
            )
            recv.wait_recv()

        out_ref[...] = jnp.sum(rbuf_ref[...], axis=0)

        for rdma in sends:
            rdma.wait_send()

    return pl.pallas_call(
        body,
        out_shape=jax.ShapeDtypeStruct((rows, n), jnp.float32),
        in_specs=[
            pl.BlockSpec(memory_space=pltpu.VMEM),
            pl.BlockSpec(memory_space=pltpu.VMEM),
        ],
        out_specs=pl.BlockSpec(memory_space=pltpu.VMEM),
        scratch_shapes=[
            pltpu.VMEM((m, n), jnp.float32),
            pltpu.VMEM((N_DEV, rows, n), jnp.float32),
            pltpu.SemaphoreType.DMA((N_DEV,)),
            pltpu.SemaphoreType.DMA((N_DEV,)),
        ],
        compiler_params=pltpu.CompilerParams(collective_id=0),
    )(A, B)


# baseline (device time: 44108 ns/iter reference)
import jax
import jax.numpy as jnp
from jax import lax
from jax.experimental import pallas as pl
from jax.experimental.pallas import tpu as pltpu

N_DEV = 32


def kernel(A, B):
    m, k = A.shape
    _, n = B.shape
    rows = m // N_DEV

    def body(a_ref, b_ref, out_ref, p_ref, rbuf_ref, send_sems, recv_sems):
        my = lax.axis_index("i")

        p_ref[...] = jnp.dot(
            a_ref[...], b_ref[...], preferred_element_type=jnp.float32
        )

        rbuf_ref[pl.ds(my, 1)] = p_ref[pl.ds(my * rows, rows), :][None]

        sends = []
        for off in range(1, N_DEV):
            d = lax.rem(my + off, N_DEV)
            rdma = pltpu.make_async_remote_copy(
                src_ref=p_ref.at[pl.ds(d * rows, rows), :],
                dst_ref=rbuf_ref.at[my],
                send_sem=send_sems.at[off],
                recv_sem=recv_sems.at[my],
                device_id=(d,),
                device_id_type=pl.DeviceIdType.MESH,
            )
            rdma.start()
            sends.append(rdma)

        for off in range(1, N_DEV):
            s = lax.rem(my + off, N_DEV)
            recv = pltpu.make_async_remote_copy(
                src_ref=p_ref.at[pl.ds(0, rows), :],
                dst_ref=rbuf_ref.at[s],
                send_sem=send_sems.at[0],
                recv_sem=recv_sems.at[s],
                device_id=(s,),
                device_id_type=pl.DeviceIdType.MESH,
            )
            recv.wait_recv()

        out_ref[...] = jnp.sum(rbuf_ref[...], axis=0)

        for rdma in sends:
            rdma.wait_send()

    return pl.pallas_call(
        body,
        out_shape=jax.ShapeDtypeStruct((rows, n), jnp.float32),
        in_specs=[
            pl.BlockSpec(memory_space=pltpu.VMEM),
            pl.BlockSpec(memory_space=pltpu.VMEM),
        ],
        out_specs=pl.BlockSpec(memory_space=pltpu.VMEM),
        scratch_shapes=[
            pltpu.VMEM((m, n), jnp.float32),
            pltpu.VMEM((N_DEV, rows, n), jnp.float32),
            pltpu.SemaphoreType.DMA((N_DEV,)),
            pltpu.SemaphoreType.DMA((N_DEV,)),
        ],
    )(A, B)


# device time: 43264 ns/iter; 1.0195x vs baseline; 1.0195x over previous
import jax
import jax.numpy as jnp
from jax import lax
from jax.experimental import pallas as pl
from jax.experimental.pallas import tpu as pltpu

N_DEV = 32
BLOCKS = 4
PER_BLOCK = N_DEV // BLOCKS


def kernel(A, B):
    m, k = A.shape
    _, n = B.shape
    rows = m // N_DEV
    brows = rows * PER_BLOCK

    def body(a_ref, b_ref, out_ref, a2_ref, p_ref, rbuf_ref, send_sems, recv_sems):
        my = lax.axis_index("i")
        base = my * rows

        a2_ref[pl.ds(0, m), :] = a_ref[...]
        a2_ref[pl.ds(m, m), :] = a_ref[...]

        sends = []
        for b in range(BLOCKS):
            p_ref[pl.ds(b * brows, brows), :] = jnp.dot(
                a2_ref[pl.ds(base + b * brows, brows), :],
                b_ref[...],
                preferred_element_type=jnp.float32,
            )
            if b == 0:
                rbuf_ref[pl.ds(0, 1)] = p_ref[pl.ds(0, rows), :][None]
            for off in range(b * PER_BLOCK, (b + 1) * PER_BLOCK):
                if off == 0:
                    continue
                d = lax.rem(my + off, N_DEV)
                slot = N_DEV - off
                rdma = pltpu.make_async_remote_copy(
                    src_ref=p_ref.at[pl.ds(off * rows, rows), :],
                    dst_ref=rbuf_ref.at[slot],
                    send_sem=send_sems.at[off],
                    recv_sem=recv_sems.at[slot],
                    device_id=(d,),
                    device_id_type=pl.DeviceIdType.MESH,
                )
                rdma.start()
                sends.append(rdma)

        acc = None
        for g in range(BLOCKS - 1, -1, -1):
            lo = g * PER_BLOCK
            for slot in range(lo + PER_BLOCK - 1, lo - 1, -1):
                if slot == 0:
                    continue
                recv = pltpu.make_async_remote_copy(
                    src_ref=p_ref.at[pl.ds(0, rows), :],
                    dst_ref=rbuf_ref.at[slot],
                    send_sem=send_sems.at[0],
                    recv_sem=recv_sems.at[slot],
                    device_id=(0,),
                    device_id_type=pl.DeviceIdType.MESH,
                )
                recv.wait_recv()
            part = jnp.sum(rbuf_ref[pl.ds(lo, PER_BLOCK)], axis=0)
            acc = part if acc is None else acc + part
        out_ref[...] = acc

        for rdma in sends:
            rdma.wait_send()

    return pl.pallas_call(
        body,
        out_shape=jax.ShapeDtypeStruct((rows, n), jnp.float32),
        in_specs=[
            pl.BlockSpec(memory_space=pltpu.VMEM),
            pl.BlockSpec(memory_space=pltpu.VMEM),
        ],
        out_specs=pl.BlockSpec(memory_space=pltpu.VMEM),
        scratch_shapes=[
            pltpu.VMEM((2 * m, k), jnp.float32),
            pltpu.VMEM((m, n), jnp.float32),
            pltpu.VMEM((N_DEV, rows, n), jnp.float32),
            pltpu.SemaphoreType.DMA((N_DEV,)),
            pltpu.SemaphoreType.DMA((N_DEV,)),
        ],
    )(A, B)


# device time: 30221 ns/iter; 1.4595x vs baseline; 1.4316x over previous
import jax
import jax.numpy as jnp
from jax import lax
from jax.experimental import pallas as pl
from jax.experimental.pallas import tpu as pltpu

N_DEV = 32
BLOCKS = 4
PER_BLOCK = N_DEV // BLOCKS


def kernel(A, B):
    m, k = A.shape
    _, n = B.shape
    rows = m // N_DEV
    brows = rows * PER_BLOCK

    def body(a_ref, b_ref, out_ref, a2_ref, b_bf_ref, p_ref, rbuf_ref,
             send_sems, recv_sems):
        my = lax.axis_index("i")
        base = my * rows

        a_bf = a_ref[...].astype(jnp.bfloat16)
        b_bf_ref[...] = b_ref[...].astype(jnp.bfloat16)
        a2_ref[pl.ds(0, m), :] = a_bf
        a2_ref[pl.ds(m, m), :] = a_bf

        sends = []
        for b in range(BLOCKS):
            p_ref[pl.ds(b * brows, brows), :] = jnp.dot(
                a2_ref[pl.ds(base + b * brows, brows), :],
                b_bf_ref[...],
                preferred_element_type=jnp.float32,
            ).astype(jnp.bfloat16)
            if b == 0:
                rbuf_ref[pl.ds(0, 1)] = p_ref[pl.ds(0, rows), :][None]
            for off in range(b * PER_BLOCK, (b + 1) * PER_BLOCK):
                if off == 0:
                    continue
                d = lax.rem(my + off, N_DEV)
                slot = N_DEV - off
                rdma = pltpu.make_async_remote_copy(
                    src_ref=p_ref.at[pl.ds(off * rows, rows), :],
                    dst_ref=rbuf_ref.at[slot],
                    send_sem=send_sems.at[off],
                    recv_sem=recv_sems.at[slot],
                    device_id=(d,),
                    device_id_type=pl.DeviceIdType.MESH,
                )
                rdma.start()
                sends.append(rdma)

        acc = None
        for g in range(BLOCKS - 1, -1, -1):
            lo = g * PER_BLOCK
            for slot in range(lo + PER_BLOCK - 1, lo - 1, -1):
                if slot == 0:
                    continue
                recv = pltpu.make_async_remote_copy(
                    src_ref=p_ref.at[pl.ds(0, rows), :],
                    dst_ref=rbuf_ref.at[slot],
                    send_sem=send_sems.at[0],
                    recv_sem=recv_sems.at[slot],
                    device_id=(0,),
                    device_id_type=pl.DeviceIdType.MESH,
                )
                recv.wait_recv()
            part = jnp.sum(
                rbuf_ref[pl.ds(lo, PER_BLOCK)].astype(jnp.float32), axis=0
            )
            acc = part if acc is None else acc + part
        out_ref[...] = acc

        for rdma in sends:
            rdma.wait_send()

    return pl.pallas_call(
        body,
        out_shape=jax.ShapeDtypeStruct((rows, n), jnp.float32),
        in_specs=[
            pl.BlockSpec(memory_space=pltpu.VMEM),
            pl.BlockSpec(memory_space=pltpu.VMEM),
        ],
        out_specs=pl.BlockSpec(memory_space=pltpu.VMEM),
        scratch_shapes=[
            pltpu.VMEM((2 * m, k), jnp.bfloat16),
            pltpu.VMEM((k, n), jnp.bfloat16),
            pltpu.VMEM((m, n), jnp.bfloat16),
            pltpu.VMEM((N_DEV, rows, n), jnp.bfloat16),
            pltpu.SemaphoreType.DMA((N_DEV,)),
            pltpu.SemaphoreType.DMA((N_DEV,)),
        ],
    )(A, B)


# device time: 23825 ns/iter; 1.8513x vs baseline; 1.2685x over previous
import jax
import jax.numpy as jnp
from jax import lax
from jax.experimental import pallas as pl
from jax.experimental.pallas import tpu as pltpu

N_DEV = 32
BLOCKS = 4
PER_BLOCK = N_DEV // BLOCKS


def kernel(A, B):
    m, k = A.shape
    _, n = B.shape
    rows = m // N_DEV
    brows = rows * PER_BLOCK

    def body(a_ref, b_ref, out_ref, a2_ref, b_bf_ref, p_ref, rbuf_ref,
             send_sems, recv_sems):
        my = lax.axis_index("i")
        base = my * rows

        barrier_sem = pltpu.get_barrier_semaphore()
        for nbr in (lax.rem(my + 1, N_DEV), lax.rem(my + N_DEV - 1, N_DEV)):
            pl.semaphore_signal(
                barrier_sem, inc=1,
                device_id=(nbr,), device_id_type=pl.DeviceIdType.MESH,
            )
        pl.semaphore_wait(barrier_sem, 2)

        a_bf = a_ref[...].astype(jnp.bfloat16)
        b_bf_ref[...] = b_ref[...].astype(jnp.bfloat16)
        a2_ref[pl.ds(0, m), :] = a_bf
        a2_ref[pl.ds(m, m), :] = a_bf

        sends = []
        for b in range(BLOCKS):
            p_ref[pl.ds(b * brows, brows), :] = jnp.dot(
                a2_ref[pl.ds(base + b * brows, brows), :],
                b_bf_ref[...],
                preferred_element_type=jnp.float32,
            ).astype(jnp.bfloat16)
            if b == 0:
                rbuf_ref[pl.ds(0, 1)] = p_ref[pl.ds(0, rows), :][None]
            for off in range(b * PER_BLOCK, (b + 1) * PER_BLOCK):
                if off == 0:
                    continue
                d = lax.rem(my + off, N_DEV)
                slot = N_DEV - off
                rdma = pltpu.make_async_remote_copy(
                    src_ref=p_ref.at[pl.ds(off * rows, rows), :],
                    dst_ref=rbuf_ref.at[slot],
                    send_sem=send_sems.at[off],
                    recv_sem=recv_sems.at[slot],
                    device_id=(d,),
                    device_id_type=pl.DeviceIdType.MESH,
                )
                rdma.start()
                sends.append(rdma)

        acc = None
        for g in range(BLOCKS - 1, -1, -1):
            lo = g * PER_BLOCK
            for slot in range(lo + PER_BLOCK - 1, lo - 1, -1):
                if slot == 0:
                    continue
                recv = pltpu.make_async_remote_copy(
                    src_ref=p_ref.at[pl.ds(0, rows), :],
                    dst_ref=rbuf_ref.at[slot],
                    send_sem=send_sems.at[0],
                    recv_sem=recv_sems.at[slot],
                    device_id=(0,),
                    device_id_type=pl.DeviceIdType.MESH,
                )
                recv.wait_recv()
            part = jnp.sum(
                rbuf_ref[pl.ds(lo, PER_BLOCK)].astype(jnp.float32), axis=0
            )
            acc = part if acc is None else acc + part
        out_ref[...] = acc

        for rdma in sends:
            rdma.wait_send()

    return pl.pallas_call(
        body,
        out_shape=jax.ShapeDtypeStruct((rows, n), jnp.float32),
        in_specs=[
            pl.BlockSpec(memory_space=pltpu.VMEM),
            pl.BlockSpec(memory_space=pltpu.VMEM),
        ],
        out_specs=pl.BlockSpec(memory_space=pltpu.VMEM),
        scratch_shapes=[
            pltpu.VMEM((2 * m, k), jnp.bfloat16),
            pltpu.VMEM((k, n), jnp.bfloat16),
            pltpu.VMEM((m, n), jnp.bfloat16),
            pltpu.VMEM((N_DEV, rows, n), jnp.bfloat16),
            pltpu.SemaphoreType.DMA((N_DEV,)),
            pltpu.SemaphoreType.DMA((N_DEV,)),
        ],
        compiler_params=pltpu.CompilerParams(collective_id=0),
    )(A, B)


# device time: 21777 ns/iter; 2.0254x vs baseline; 1.0940x over previous
import jax
import jax.numpy as jnp
from jax import lax
from jax.experimental import pallas as pl
from jax.experimental.pallas import tpu as pltpu

N_DEV = 32
BLOCKS = 4
PER_BLOCK = N_DEV // BLOCKS


def kernel(A, B):
    m, k = A.shape
    _, n = B.shape
    rows = m // N_DEV
    brows = rows * PER_BLOCK

    def body(a_ref, b_ref, out_ref, a2_ref, b_bf_ref, p_ref, rbuf_ref,
             send_sems, recv_sems):
        my = lax.axis_index("i")
        base = my * rows

        barrier_sem = pltpu.get_barrier_semaphore()
        for peer_off in range(1, N_DEV):
            pl.semaphore_signal(
                barrier_sem, inc=1,
                device_id=(lax.rem(my + peer_off, N_DEV),),
                device_id_type=pl.DeviceIdType.MESH,
            )

        a_bf = a_ref[...].astype(jnp.bfloat16)
        b_bf_ref[...] = b_ref[...].astype(jnp.bfloat16)
        a2_ref[pl.ds(0, m), :] = a_bf
        a2_ref[pl.ds(m, m), :] = a_bf

        sends = []
        for b in range(BLOCKS):
            p_ref[pl.ds(b * brows, brows), :] = jnp.dot(
                a2_ref[pl.ds(base + b * brows, brows), :],
                b_bf_ref[...],
                preferred_element_type=jnp.float32,
            ).astype(jnp.bfloat16)
            if b == 0:
                rbuf_ref[pl.ds(0, 1)] = p_ref[pl.ds(0, rows), :][None]
                pl.semaphore_wait(barrier_sem, N_DEV - 1)
            for off in range(b * PER_BLOCK, (b + 1) * PER_BLOCK):
                if off == 0:
                    continue
                d = lax.rem(my + off, N_DEV)
                slot = N_DEV - off
                rdma = pltpu.make_async_remote_copy(
                    src_ref=p_ref.at[pl.ds(off * rows, rows), :],
                    dst_ref=rbuf_ref.at[slot],
                    send_sem=send_sems.at[off],
                    recv_sem=recv_sems.at[slot],
                    device_id=(d,),
                    device_id_type=pl.DeviceIdType.MESH,
                )
                rdma.start()
                sends.append(rdma)

        acc = None
        for g in range(BLOCKS - 1, -1, -1):
            lo = g * PER_BLOCK
            for slot in range(lo + PER_BLOCK - 1, lo - 1, -1):
                if slot == 0:
                    continue
                recv = pltpu.make_async_remote_copy(
                    src_ref=p_ref.at[pl.ds(0, rows), :],
                    dst_ref=rbuf_ref.at[slot],
                    send_sem=send_sems.at[0],
                    recv_sem=recv_sems.at[slot],
                    device_id=(0,),
                    device_id_type=pl.DeviceIdType.MESH,
                )
                recv.wait_recv()
            part = jnp.sum(
                rbuf_ref[pl.ds(lo, PER_BLOCK)].astype(jnp.float32), axis=0
            )
            acc = part if acc is None else acc + part
        out_ref[...] = acc

        for rdma in sends:
            rdma.wait_send()

    return pl.pallas_call(
        body,
        out_shape=jax.ShapeDtypeStruct((rows, n), jnp.float32),
        in_specs=[
            pl.BlockSpec(memory_space=pltpu.VMEM),
            pl.BlockSpec(memory_space=pltpu.VMEM),
        ],
        out_specs=pl.BlockSpec(memory_space=pltpu.VMEM),
        scratch_shapes=[
            pltpu.VMEM((2 * m, k), jnp.bfloat16),
            pltpu.VMEM((k, n), jnp.bfloat16),
            pltpu.VMEM((m, n), jnp.bfloat16),
            pltpu.VMEM((N_DEV, rows, n), jnp.bfloat16),
            pltpu.SemaphoreType.DMA((N_DEV,)),
            pltpu.SemaphoreType.DMA((N_DEV,)),
        ],
        compiler_params=pltpu.CompilerParams(collective_id=0),
    )(A, B)
